# baseline (speedup 1.0000x reference)
import functools

import jax
import jax.numpy as jnp
from jax import lax
from jax.experimental import pallas as pl
from jax.experimental.pallas import tpu as pltpu
from jax.experimental.pallas import tpu_sc as plsc

_LANES = 16
_NC = 2
_NS = 16
_NW = _NC * _NS

_LP = 208
_NBUF = 2


def _embbag_sc(idx_pad, table2):
    B, LP = idx_pad.shape
    E = table2.shape[1] // 2
    SPT = B // _NW
    NCH = E // _LANES

    mesh = plsc.VectorSubcoreMesh(
        core_axis_name="c", subcore_axis_name="s",
        num_cores=_NC, num_subcores=_NS)

    @functools.partial(
        pl.kernel,
        mesh=mesh,
        out_type=jax.ShapeDtypeStruct((B, E), jnp.float32),
        scratch_types=[
            pltpu.VMEM((SPT, LP), jnp.int32),
            pltpu.VMEM((_NBUF, LP, 2 * E), jnp.float32),
            pltpu.VMEM((SPT, E), jnp.float32),
        ] + [pltpu.SemaphoreType.DMA] * _NBUF,
        compiler_params=pltpu.CompilerParams(needs_layout_passes=False),
    )
    def body(idx_hbm, table_hbm, pooled_hbm, idx_v, rows_v, pool_v, *sems):
        wid = lax.axis_index("s") * _NC + lax.axis_index("c")
        base = wid * SPT
        pltpu.sync_copy(idx_hbm.at[pl.ds(base, SPT)], idx_v)

        def fire(s, b):
            for k in range(LP // _LANES):
                iv = idx_v[s, pl.ds(k * _LANES, _LANES)]
                pltpu.async_copy(
                    table_hbm.at[lax.shift_right_logical(iv, 1)],
                    rows_v.at[b, pl.ds(k * _LANES, _LANES)], sems[b])

        def wait(b):
            pltpu.make_async_copy(
                table_hbm.at[pl.ds(0, LP)], rows_v.at[b], sems[b]).wait()

        for b in range(_NBUF):
            fire(b, b)

        fzero = jnp.zeros((_LANES,), jnp.float32)
        ione = jnp.ones((_LANES,), jnp.int32)
        izero = jnp.zeros((_LANES,), jnp.int32)

        def group(g, carry):
            for b in range(_NBUF):
                s = g * _NBUF + b
                wait(b)

                def kgroup(k, accs):
                    pv = idx_v[s, pl.ds(k * _LANES, _LANES)] & 1
                    out = list(accs)
                    for t in range(_LANES):
                        pm = jnp.full((_LANES,), pv[t]) != 0
                        j = k * _LANES + t
                        for c in range(NCH):
                            lo = rows_v[b, j, pl.ds(c * _LANES, _LANES)]
                            hi = rows_v[b, j, pl.ds(E + c * _LANES, _LANES)]
                            out[c] = out[c] + jnp.where(pm, hi, lo)
                    return tuple(out)

                accs = lax.fori_loop(0, LP // _LANES, kgroup, (fzero,) * NCH)

                ns = s + _NBUF

                @pl.when(ns < SPT)
                def _():
                    fire(ns, b)

                def cbody(k, cv):
                    iv = idx_v[s, pl.ds(k * _LANES, _LANES)]
                    return cv + jnp.where(iv != 0, ione, izero)

                cv = lax.fori_loop(0, LP // _LANES, cbody, izero)
                cnt = jnp.maximum(jnp.sum(cv), 1)
                cntf = jnp.full((_LANES,), cnt.astype(jnp.float32))
                for c in range(NCH):
                    pool_v[s, pl.ds(c * _LANES, _LANES)] = accs[c] / cntf
            return carry

        lax.fori_loop(0, SPT // _NBUF, group, 0)
        pltpu.sync_copy(pool_v, pooled_hbm.at[pl.ds(base, SPT)])

    return body(idx_pad, table2)


def _mlp_tc(pooled, W1, b1, W2, b2):
    B, E = pooled.shape
    H = W1.shape[1]
    C = W2.shape[1]
    BT = 512

    def body(x_ref, w1_ref, b1_ref, w2_ref, b2_ref, o_ref):
        x = x_ref[...]
        h = jnp.dot(x, w1_ref[...], preferred_element_type=jnp.float32)
        h = jnp.maximum(h + b1_ref[...], 0.0)
        logits = jnp.dot(h, w2_ref[...], preferred_element_type=jnp.float32)
        logits = logits + b2_ref[...]
        m = jnp.max(logits, axis=1, keepdims=True)
        e = jnp.exp(logits - m)
        o_ref[...] = e / jnp.sum(e, axis=1, keepdims=True)

    return pl.pallas_call(
        body,
        grid=(B // BT,),
        in_specs=[
            pl.BlockSpec((BT, E), lambda i: (i, 0)),
            pl.BlockSpec((E, H), lambda i: (0, 0)),
            pl.BlockSpec((1, H), lambda i: (0, 0)),
            pl.BlockSpec((H, C), lambda i: (0, 0)),
            pl.BlockSpec((1, C), lambda i: (0, 0)),
        ],
        out_specs=pl.BlockSpec((BT, C), lambda i: (i, 0)),
        out_shape=jax.ShapeDtypeStruct((B, C), jnp.float32),
    )(pooled, W1, b1.reshape(1, H), W2, b2.reshape(1, C))


def kernel(batch_inputs, batch_lengths, emb_table, W1, b1, W2, b2):
    B, L = batch_inputs.shape
    V, E = emb_table.shape
    idx_pad = jnp.pad(batch_inputs, ((0, 0), (0, _LP - L)))
    table2 = emb_table.reshape(V // 2, 2 * E)
    pooled = _embbag_sc(idx_pad, table2)
    return _mlp_tc(pooled, W1, b1, W2, b2)


# speedup vs baseline: 1.6869x; 1.6869x over previous
import functools

import jax
import jax.numpy as jnp
import numpy as _np
from jax import lax
from jax.experimental import pallas as pl
from jax.experimental.pallas import tpu as pltpu
from jax.experimental.pallas import tpu_sc as plsc

_LANES = 16
_NC = 2
_NS = 16
_NW = _NC * _NS

_LP = 208
_NBUF = 4

_UNPACK_PERM = _np.concatenate(
    [_np.concatenate([_np.arange(0, 32, 2), _np.arange(1, 32, 2)]) + 32 * c
     for c in range(2)])


def _embbag_sc(idx_pad, table_bf):
    B, LP = idx_pad.shape
    E = table_bf.shape[1]
    SPT = B // _NW
    NCH = E // 32

    mesh = plsc.VectorSubcoreMesh(
        core_axis_name="c", subcore_axis_name="s",
        num_cores=_NC, num_subcores=_NS)

    @functools.partial(
        pl.kernel,
        mesh=mesh,
        out_type=jax.ShapeDtypeStruct((B, E), jnp.float32),
        scratch_types=[
            pltpu.VMEM((SPT, LP), jnp.int32),
            pltpu.VMEM((_NBUF, LP, E), jnp.bfloat16),
            pltpu.VMEM((SPT, E), jnp.float32),
        ] + [pltpu.SemaphoreType.DMA] * _NBUF,
        compiler_params=pltpu.CompilerParams(
            use_tc_tiling_on_sc=False, needs_layout_passes=False),
    )
    def body(idx_hbm, table_hbm, pooled_hbm, idx_v, rows_v, pool_v, *sems):
        wid = lax.axis_index("s") * _NC + lax.axis_index("c")
        base = wid * SPT
        pltpu.sync_copy(idx_hbm.at[pl.ds(base, SPT)], idx_v)

        def fire(s, b):
            pltpu.async_copy(
                table_hbm.at[idx_v.at[s]], rows_v.at[b], sems[b])

        def wait(b):
            pltpu.make_async_copy(
                table_hbm.at[pl.ds(0, LP)], rows_v.at[b], sems[b]).wait()

        for b in range(_NBUF):
            fire(b, b)

        fzero = jnp.zeros((_LANES,), jnp.float32)
        ione = jnp.ones((_LANES,), jnp.int32)
        izero = jnp.zeros((_LANES,), jnp.int32)

        def group(g, carry):
            for b in range(_NBUF):
                s = g * _NBUF + b
                wait(b)

                def jbody(j, accs):
                    out = list(accs)
                    for c in range(NCH):
                        x = rows_v[b, j, pl.ds(c * 32, 32)]
                        lo, hi = plsc.unpack(
                            x, format=plsc.PackFormat.INTERLEAVED,
                            preferred_element_type=jnp.float32)
                        out[2 * c] = out[2 * c] + lo
                        out[2 * c + 1] = out[2 * c + 1] + hi
                    return tuple(out)

                accs = lax.fori_loop(0, LP, jbody, (fzero,) * (2 * NCH))

                ns = s + _NBUF

                @pl.when(ns < SPT)
                def _():
                    fire(ns, b)

                def cbody(k, cv):
                    iv = idx_v[s, pl.ds(k * _LANES, _LANES)]
                    return cv + jnp.where(iv != 0, ione, izero)

                cv = lax.fori_loop(0, LP // _LANES, cbody, izero)
                cnt = jnp.maximum(jnp.sum(cv), 1)
                cntf = jnp.full((_LANES,), cnt.astype(jnp.float32))
                for c in range(2 * NCH):
                    pool_v[s, pl.ds(c * _LANES, _LANES)] = accs[c] / cntf
            return carry

        lax.fori_loop(0, SPT // _NBUF, group, 0)
        pltpu.sync_copy(pool_v, pooled_hbm.at[pl.ds(base, SPT)])

    return body(idx_pad, table_bf)


def _mlp_tc(pooled, W1, b1, W2, b2):
    B, E = pooled.shape
    H = W1.shape[1]
    C = W2.shape[1]
    BT = 512

    def body(x_ref, w1_ref, b1_ref, w2_ref, b2_ref, o_ref):
        x = x_ref[...]
        h = jnp.dot(x, w1_ref[...], preferred_element_type=jnp.float32)
        h = jnp.maximum(h + b1_ref[...], 0.0)
        logits = jnp.dot(h, w2_ref[...], preferred_element_type=jnp.float32)
        logits = logits + b2_ref[...]
        m = jnp.max(logits, axis=1, keepdims=True)
        e = jnp.exp(logits - m)
        o_ref[...] = e / jnp.sum(e, axis=1, keepdims=True)

    return pl.pallas_call(
        body,
        grid=(B // BT,),
        in_specs=[
            pl.BlockSpec((BT, E), lambda i: (i, 0)),
            pl.BlockSpec((E, H), lambda i: (0, 0)),
            pl.BlockSpec((1, H), lambda i: (0, 0)),
            pl.BlockSpec((H, C), lambda i: (0, 0)),
            pl.BlockSpec((1, C), lambda i: (0, 0)),
        ],
        out_specs=pl.BlockSpec((BT, C), lambda i: (i, 0)),
        out_shape=jax.ShapeDtypeStruct((B, C), jnp.float32),
    )(pooled, W1, b1.reshape(1, H), W2, b2.reshape(1, C))


def kernel(batch_inputs, batch_lengths, emb_table, W1, b1, W2, b2):
    B, L = batch_inputs.shape
    idx_pad = jnp.pad(batch_inputs, ((0, 0), (0, _LP - L)))
    table_bf = emb_table.astype(jnp.bfloat16)
    pooled = _embbag_sc(idx_pad, table_bf)
    W1p = W1[_UNPACK_PERM, :]
    return _mlp_tc(pooled, W1p, b1, W2, b2)
